# baseline (speedup 1.0000x reference)
import jax
import jax.numpy as jnp
from jax.experimental import pallas as pl
from jax.experimental.pallas import tpu as pltpu
from jax.experimental.pallas import tpu_sc as plsc

_BW = 128
_TC = 4096


def _linearize_table(weight):
    rows, dim = weight.shape
    out_rows = rows * dim // 128
    blk_out = _TC * dim // 128
    grid = (rows + _TC - 1) // _TC

    group = 128 // dim

    def tbody(i_ref, o_ref, s_ref):
        s_ref[...] = i_ref[...].T
        for a in range(group):
            o_ref[:, dim * a : dim * (a + 1)] = s_ref[a::group, :]

    return pl.pallas_call(
        tbody,
        grid=(grid,),
        in_specs=[pl.BlockSpec((dim, _TC), lambda i: (0, i))],
        out_specs=pl.BlockSpec((blk_out, 128), lambda i: (i, 0)),
        out_shape=jax.ShapeDtypeStruct((out_rows, 128), weight.dtype),
        scratch_shapes=[pltpu.VMEM((_TC, dim), weight.dtype)],
    )(weight.T)


def kernel(token_ids, weight):
    batch, hist = token_ids.shape
    dim = weight.shape[1]
    mesh = plsc.VectorSubcoreMesh(core_axis_name="c", subcore_axis_name="s")
    weight = _linearize_table(weight).reshape(weight.shape)

    idx_t = token_ids.T

    @pl.kernel(
        out_type=jax.ShapeDtypeStruct((hist, dim, batch), weight.dtype),
        mesh=mesh,
        scratch_types=[
            pltpu.VMEM((_BW, weight.shape[1]), weight.dtype),
            pltpu.SemaphoreType.DMA,
        ],
        compiler_params=pltpu.CompilerParams(
            use_tc_tiling_on_sc=False, needs_layout_passes=False
        ),
    )
    def gather_kernel(w_hbm, i_hbm, o_hbm, t_vmem, sem):
        def body(i_vmem, o_vmem):
            pltpu.sync_copy(w_hbm.at[i_vmem.at[0]], t_vmem)
            ot = o_vmem.at[0]

            @pl.loop(0, dim)
            def _(d):
                cols = jnp.zeros((16,), jnp.int32) + d
                for g in range(_BW // 16):
                    rows = jax.lax.iota(jnp.int32, 16) + (16 * g)
                    vals = plsc.load_gather(t_vmem, [rows, cols])
                    ot[d, pl.ds(16 * g, 16)] = vals

        pltpu.emit_pipeline(
            body,
            grid=(hist, batch // _BW),
            in_specs=[pl.BlockSpec((1, _BW), index_map=lambda h, j: (h, j))],
            out_specs=[
                pl.BlockSpec((1, dim, _BW), index_map=lambda h, j: (h, 0, j))
            ],
            core_axis_name=("c", "s"),
            dimension_semantics=(pltpu.PARALLEL, pltpu.PARALLEL),
        )(i_hbm, o_hbm)

    return gather_kernel(weight, idx_t).transpose(2, 0, 1)


# speedup vs baseline: 1.4418x; 1.4418x over previous
import jax
import jax.numpy as jnp
from jax.experimental import pallas as pl
from jax.experimental.pallas import tpu as pltpu
from jax.experimental.pallas import tpu_sc as plsc

_K = 16
_TC = 8192


def _linearize_table(weight):
    rows, dim = weight.shape
    out_rows = rows * dim // 128
    blk_out = _TC * dim // 128
    grid = (rows + _TC - 1) // _TC

    group = 128 // dim

    def tbody(i_ref, o_ref, s_ref):
        s_ref[...] = i_ref[...].T
        for a in range(group):
            o_ref[:, dim * a : dim * (a + 1)] = s_ref[a::group, :]

    return pl.pallas_call(
        tbody,
        grid=(grid,),
        in_specs=[pl.BlockSpec((dim, _TC), lambda i: (0, i))],
        out_specs=pl.BlockSpec((blk_out, 128), lambda i: (i, 0)),
        out_shape=jax.ShapeDtypeStruct((out_rows, 128), weight.dtype),
        scratch_shapes=[pltpu.VMEM((_TC, dim), weight.dtype)],
    )(weight.T)


def kernel(token_ids, weight):
    batch, hist = token_ids.shape
    dim = weight.shape[1]
    mesh = plsc.VectorSubcoreMesh(core_axis_name="c", subcore_axis_name="s")
    weight = _linearize_table(weight).reshape(weight.shape)

    @pl.kernel(
        out_type=jax.ShapeDtypeStruct((batch, hist, dim), weight.dtype),
        mesh=mesh,
        scratch_types=[pltpu.SemaphoreType.DMA],
        compiler_params=pltpu.CompilerParams(use_tc_tiling_on_sc=False),
    )
    def gather_kernel(w_hbm, i_hbm, o_hbm, sem):
        def body(i_vmem, o_vmem):
            copies = [
                pltpu.async_copy(
                    w_hbm.at[i_vmem.at[j]],
                    o_vmem.at[j],
                    sem,
                )
                for j in range(_K)
            ]
            for c in copies:
                c.wait()

        pltpu.emit_pipeline(
            body,
            grid=(batch // _K,),
            in_specs=[pl.BlockSpec((_K, hist), index_map=lambda i: (i, 0))],
            out_specs=[
                pl.BlockSpec((_K, hist, dim), index_map=lambda i: (i, 0, 0))
            ],
            core_axis_name=("c", "s"),
            dimension_semantics=(pltpu.PARALLEL,),
        )(i_hbm, o_hbm)

    return gather_kernel(weight, token_ids)
